# baseline (speedup 1.0000x reference)
import jax
import jax.numpy as jnp
from jax import lax
from jax.experimental import pallas as pl
from jax.experimental.pallas import tpu as pltpu

_ALPHA, _GAMMA = 0.25, 2.0
_W_CLASS, _W_BBOX, _W_GIOU = 2.0, 5.0, 2.0


def _matcher_kernel(logits_ref, boxes_ref, tgt_ref, ign_ref, imgsz_ref,
                    ids_ref, c_ref, ioa_ref):
    logits = logits_ref[0]
    boxes = boxes_ref[0]
    tgt = tgt_ref[0]
    ign = ign_ref[0]
    ids = ids_ref[0]

    q, nc = logits.shape
    t = tgt.shape[0]

    p = jax.nn.sigmoid(logits)
    one_m_p = 1.0 - p
    pos = (-_ALPHA) * jnp.log(p + 1e-8) * (one_m_p * one_m_p)
    neg = (-(1.0 - _ALPHA)) * jnp.log(one_m_p + 1e-8) * (p * p)
    cls_cost = pos - neg
    iota_c = lax.broadcasted_iota(jnp.int32, (nc, t), 0)
    onehot = jnp.where(iota_c == ids, _W_CLASS, 0.0)
    cost_class = jnp.dot(cls_cost, onehot,
                         preferred_element_type=jnp.float32)

    eye4 = jnp.where(
        lax.broadcasted_iota(jnp.int32, (4, 4), 0)
        == lax.broadcasted_iota(jnp.int32, (4, 4), 1), 1.0, 0.0)
    dn = (((1,), (1,)), ((), ()))
    tgtT = lax.dot_general(eye4, tgt, dn,
                           precision=lax.Precision.HIGHEST,
                           preferred_element_type=jnp.float32)
    ignT = lax.dot_general(eye4, ign, dn,
                           precision=lax.Precision.HIGHEST,
                           preferred_element_type=jnp.float32)

    tx1 = tgtT[0:1, :]
    ty1 = tgtT[1:2, :]
    tx2 = tgtT[2:3, :]
    ty2 = tgtT[3:4, :]

    x1 = boxes[:, 0:1]
    y1 = boxes[:, 1:2]
    x2 = boxes[:, 2:3]
    y2 = boxes[:, 3:4]
    x1b = jnp.broadcast_to(x1, (q, t))
    y1b = jnp.broadcast_to(y1, (q, t))
    x2b = jnp.broadcast_to(x2, (q, t))
    y2b = jnp.broadcast_to(y2, (q, t))

    mxx1 = jnp.maximum(x1b, tx1)
    mnx1 = jnp.minimum(x1b, tx1)
    mxx2 = jnp.maximum(x2b, tx2)
    mnx2 = jnp.minimum(x2b, tx2)
    mxy1 = jnp.maximum(y1b, ty1)
    mny1 = jnp.minimum(y1b, ty1)
    mxy2 = jnp.maximum(y2b, ty2)
    mny2 = jnp.minimum(y2b, ty2)

    w = imgsz_ref[0, 0, 0]
    h = imgsz_ref[0, 0, 1]
    sw = _W_BBOX / w
    sh = _W_BBOX / h
    cost_bbox_w = (mxx1 - mnx1) + (mxx2 - mnx2)
    cost_bbox_h = (mxy1 - mny1) + (mxy2 - mny2)

    iw = jnp.maximum(mnx2 - mxx1, 0.0)
    ih = jnp.maximum(mny2 - mxy1, 0.0)
    inter = iw * ih
    area_p = (x2 - x1) * (y2 - y1)
    area_t = (tx2 - tx1) * (ty2 - ty1)
    area_sum = area_p + area_t
    union = area_sum - inter
    enc = (mxx2 - mnx1) * (mxy2 - mny1)
    iou = inter / union
    rest = (enc - union) / enc
    c_ref[0] = (cost_class + sw * cost_bbox_w + sh * cost_bbox_h
                + _W_GIOU * (rest - iou))

    ix1 = ignT[0:1, :]
    iy1 = ignT[1:2, :]
    ix2 = ignT[2:3, :]
    iy2 = ignT[3:4, :]
    iiw = jnp.maximum(jnp.minimum(x2, ix2) - jnp.maximum(x1, ix1), 0.0)
    iih = jnp.maximum(jnp.minimum(y2, iy2) - jnp.maximum(y1, iy1), 0.0)
    inv_area = 1.0 / area_p
    ioa_ref[0] = (iiw * iih) * inv_area


def kernel(pred_logits, pred_boxes, tgt_bbox, ign_bbox, image_size_xyxy,
           image_size_xyxy_tgt, tgt_ids, *, interpret=False):
    del image_size_xyxy_tgt
    bs, q, nc = pred_logits.shape
    t = tgt_bbox.shape[0] // bs
    ni = ign_bbox.shape[0] // bs

    tgt3 = tgt_bbox.reshape(bs, t, 4)
    ign3 = ign_bbox.reshape(bs, ni, 4)
    imgsz = image_size_xyxy.reshape(bs, 1, 4)
    ids = tgt_ids.reshape(bs, 1, t).astype(jnp.int32)

    c_diag, ioa_diag = pl.pallas_call(
        _matcher_kernel,
        grid=(bs,),
        in_specs=[
            pl.BlockSpec((1, q, nc), lambda b: (b, 0, 0)),
            pl.BlockSpec((1, q, 4), lambda b: (b, 0, 0)),
            pl.BlockSpec((1, t, 4), lambda b: (b, 0, 0)),
            pl.BlockSpec((1, ni, 4), lambda b: (b, 0, 0)),
            pl.BlockSpec((1, 1, 4), lambda b: (b, 0, 0)),
            pl.BlockSpec((1, 1, t), lambda b: (b, 0, 0)),
        ],
        out_specs=[
            pl.BlockSpec((1, q, t), lambda b: (b, 0, 0)),
            pl.BlockSpec((1, q, ni), lambda b: (b, 0, 0)),
        ],
        out_shape=[
            jax.ShapeDtypeStruct((bs, q, t), jnp.float32),
            jax.ShapeDtypeStruct((bs, q, ni), jnp.float32),
        ],
        compiler_params=pltpu.CompilerParams(
            dimension_semantics=("parallel",),
        ),
        name="hungarian_matcher_cost",
        interpret=interpret,
    )(pred_logits, pred_boxes, tgt3, ign3, imgsz, ids)
    return c_diag, ioa_diag


# speedup vs baseline: 1.4151x; 1.4151x over previous
import jax
import jax.numpy as jnp
from jax import lax
from jax.experimental import pallas as pl
from jax.experimental.pallas import tpu as pltpu

_ALPHA, _GAMMA = 0.25, 2.0
_W_CLASS, _W_BBOX, _W_GIOU = 2.0, 5.0, 2.0


def _matcher_kernel(logitsT_ref, boxesT_ref, tgt_ref, ign_ref, imgsz_ref,
                    ids_ref, ct_ref, ioat_ref):
    logitsT = logitsT_ref[0]
    boxesT = boxesT_ref[0]
    tgt = tgt_ref[0]
    ign = ign_ref[0]
    ids = ids_ref[0]

    nc, qn = logitsT.shape
    t = tgt.shape[0]

    p = jax.nn.sigmoid(logitsT)
    one_m_p = 1.0 - p
    pos = (-_ALPHA) * jnp.log(p + 1e-8) * (one_m_p * one_m_p)
    neg = (-(1.0 - _ALPHA)) * jnp.log(one_m_p + 1e-8) * (p * p)
    cls_cost = pos - neg
    iota_c = lax.broadcasted_iota(jnp.int32, (nc, t), 0)
    onehot = jnp.where(iota_c == ids, _W_CLASS, 0.0)
    cost_class = lax.dot_general(onehot, cls_cost,
                                 (((0,), (0,)), ((), ())),
                                 preferred_element_type=jnp.float32)

    x1 = boxesT[0:1, :]
    y1 = boxesT[1:2, :]
    x2 = boxesT[2:3, :]
    y2 = boxesT[3:4, :]
    tx1 = jnp.broadcast_to(tgt[:, 0:1], (t, qn))
    ty1 = jnp.broadcast_to(tgt[:, 1:2], (t, qn))
    tx2 = jnp.broadcast_to(tgt[:, 2:3], (t, qn))
    ty2 = jnp.broadcast_to(tgt[:, 3:4], (t, qn))

    mxx1 = jnp.maximum(tx1, x1)
    mnx1 = jnp.minimum(tx1, x1)
    mxx2 = jnp.maximum(tx2, x2)
    mnx2 = jnp.minimum(tx2, x2)
    mxy1 = jnp.maximum(ty1, y1)
    mny1 = jnp.minimum(ty1, y1)
    mxy2 = jnp.maximum(ty2, y2)
    mny2 = jnp.minimum(ty2, y2)

    w = imgsz_ref[0, 0, 0]
    h = imgsz_ref[0, 0, 1]
    sw = _W_BBOX / w
    sh = _W_BBOX / h
    cost_bbox_w = (mxx1 - mnx1) + (mxx2 - mnx2)
    cost_bbox_h = (mxy1 - mny1) + (mxy2 - mny2)

    iw = jnp.maximum(mnx2 - mxx1, 0.0)
    ih = jnp.maximum(mny2 - mxy1, 0.0)
    inter = iw * ih
    area_p = (x2 - x1) * (y2 - y1)
    tw = tgt[:, 2:3] - tgt[:, 0:1]
    th = tgt[:, 3:4] - tgt[:, 1:2]
    area_t = tw * th
    area_sum = area_t + area_p
    union = area_sum - inter
    enc = (mxx2 - mnx1) * (mxy2 - mny1)
    iou = inter / union
    rest = (enc - union) / enc
    ct_ref[0] = (cost_class + sw * cost_bbox_w + sh * cost_bbox_h
                 + _W_GIOU * (rest - iou))

    ix1 = ign[:, 0:1]
    iy1 = ign[:, 1:2]
    ix2 = ign[:, 2:3]
    iy2 = ign[:, 3:4]
    iiw = jnp.maximum(jnp.minimum(ix2, x2) - jnp.maximum(ix1, x1), 0.0)
    iih = jnp.maximum(jnp.minimum(iy2, y2) - jnp.maximum(iy1, y1), 0.0)
    inv_area = 1.0 / area_p
    ioat_ref[0] = (iiw * iih) * inv_area


def kernel(pred_logits, pred_boxes, tgt_bbox, ign_bbox, image_size_xyxy,
           image_size_xyxy_tgt, tgt_ids, *, interpret=False):
    del image_size_xyxy_tgt
    bs, q, nc = pred_logits.shape
    t = tgt_bbox.shape[0] // bs
    ni = ign_bbox.shape[0] // bs

    logitsT = pred_logits.transpose(0, 2, 1)
    boxesT = pred_boxes.transpose(0, 2, 1)
    tgt3 = tgt_bbox.reshape(bs, t, 4)
    ign3 = ign_bbox.reshape(bs, ni, 4)
    imgsz = image_size_xyxy.reshape(bs, 1, 4)
    ids = tgt_ids.reshape(bs, 1, t).astype(jnp.int32)

    ct, ioat = pl.pallas_call(
        _matcher_kernel,
        grid=(bs,),
        in_specs=[
            pl.BlockSpec((1, nc, q), lambda b: (b, 0, 0)),
            pl.BlockSpec((1, 4, q), lambda b: (b, 0, 0)),
            pl.BlockSpec((1, t, 4), lambda b: (b, 0, 0)),
            pl.BlockSpec((1, ni, 4), lambda b: (b, 0, 0)),
            pl.BlockSpec((1, 1, 4), lambda b: (b, 0, 0)),
            pl.BlockSpec((1, 1, t), lambda b: (b, 0, 0)),
        ],
        out_specs=[
            pl.BlockSpec((1, t, q), lambda b: (b, 0, 0)),
            pl.BlockSpec((1, ni, q), lambda b: (b, 0, 0)),
        ],
        out_shape=[
            jax.ShapeDtypeStruct((bs, t, q), jnp.float32),
            jax.ShapeDtypeStruct((bs, ni, q), jnp.float32),
        ],
        compiler_params=pltpu.CompilerParams(
            dimension_semantics=("parallel",),
        ),
        name="hungarian_matcher_cost",
        interpret=interpret,
    )(logitsT, boxesT, tgt3, ign3, imgsz, ids)
    return ct.transpose(0, 2, 1), ioat.transpose(0, 2, 1)


# speedup vs baseline: 1.4461x; 1.0219x over previous
import jax
import jax.numpy as jnp
from jax import lax
from jax.experimental import pallas as pl
from jax.experimental.pallas import tpu as pltpu

_ALPHA, _GAMMA = 0.25, 2.0
_W_CLASS, _W_BBOX, _W_GIOU = 2.0, 5.0, 2.0


def _matcher_kernel(logitsT_ref, boxesT_ref, pack_ref, ids_ref,
                    ct_ref, ioat_ref):
    logitsT = logitsT_ref[0]
    boxesT = boxesT_ref[0]
    ids = ids_ref[0]

    nc, qn = logitsT.shape
    t = ids.shape[1]

    pack = pack_ref[0]
    tgt = pack[0:t, :]
    ign = pack[t + 4:, :]

    p = jax.nn.sigmoid(logitsT)
    one_m_p = 1.0 - p
    pos = (-_ALPHA) * jnp.log(p + 1e-8) * (one_m_p * one_m_p)
    neg = (-(1.0 - _ALPHA)) * jnp.log(one_m_p + 1e-8) * (p * p)
    cls_cost = pos - neg
    iota_c = lax.broadcasted_iota(jnp.int32, (nc, t), 0)
    onehot = jnp.where(iota_c == ids, _W_CLASS, 0.0)
    cost_class = lax.dot_general(onehot, cls_cost,
                                 (((0,), (0,)), ((), ())),
                                 preferred_element_type=jnp.float32)

    x1 = boxesT[0:1, :]
    y1 = boxesT[1:2, :]
    x2 = boxesT[2:3, :]
    y2 = boxesT[3:4, :]
    tx1 = jnp.broadcast_to(tgt[:, 0:1], (t, qn))
    ty1 = jnp.broadcast_to(tgt[:, 1:2], (t, qn))
    tx2 = jnp.broadcast_to(tgt[:, 2:3], (t, qn))
    ty2 = jnp.broadcast_to(tgt[:, 3:4], (t, qn))

    mxx1 = jnp.maximum(tx1, x1)
    mnx1 = jnp.minimum(tx1, x1)
    mxx2 = jnp.maximum(tx2, x2)
    mnx2 = jnp.minimum(tx2, x2)
    mxy1 = jnp.maximum(ty1, y1)
    mny1 = jnp.minimum(ty1, y1)
    mxy2 = jnp.maximum(ty2, y2)
    mny2 = jnp.minimum(ty2, y2)

    w = pack_ref[0, t, 0]
    h = pack_ref[0, t, 1]
    sw = _W_BBOX / w
    sh = _W_BBOX / h
    cost_bbox_w = (mxx1 - mnx1) + (mxx2 - mnx2)
    cost_bbox_h = (mxy1 - mny1) + (mxy2 - mny2)

    iw = jnp.maximum(mnx2 - mxx1, 0.0)
    ih = jnp.maximum(mny2 - mxy1, 0.0)
    inter = iw * ih
    area_p = (x2 - x1) * (y2 - y1)
    tw = tgt[:, 2:3] - tgt[:, 0:1]
    th = tgt[:, 3:4] - tgt[:, 1:2]
    area_t = tw * th
    area_sum = area_t + area_p
    union = area_sum - inter
    enc = (mxx2 - mnx1) * (mxy2 - mny1)
    iou = inter / union
    rest = (enc - union) / enc
    ct_ref[0] = (cost_class + sw * cost_bbox_w + sh * cost_bbox_h
                 + _W_GIOU * (rest - iou))

    ix1 = ign[:, 0:1]
    iy1 = ign[:, 1:2]
    ix2 = ign[:, 2:3]
    iy2 = ign[:, 3:4]
    iiw = jnp.maximum(jnp.minimum(ix2, x2) - jnp.maximum(ix1, x1), 0.0)
    iih = jnp.maximum(jnp.minimum(iy2, y2) - jnp.maximum(iy1, y1), 0.0)
    inv_area = 1.0 / area_p
    ioat_ref[0] = (iiw * iih) * inv_area


def kernel(pred_logits, pred_boxes, tgt_bbox, ign_bbox, image_size_xyxy,
           image_size_xyxy_tgt, tgt_ids, *, interpret=False):
    del image_size_xyxy_tgt
    bs, q, nc = pred_logits.shape
    t = tgt_bbox.shape[0] // bs
    ni = ign_bbox.shape[0] // bs

    logitsT = pred_logits.transpose(0, 2, 1)
    boxesT = pred_boxes.transpose(0, 2, 1)
    npack = t + 4 + ni
    pack = jnp.concatenate([
        tgt_bbox.reshape(bs, t, 4),
        image_size_xyxy.reshape(bs, 1, 4),
        jnp.zeros((bs, 3, 4), jnp.float32),
        ign_bbox.reshape(bs, ni, 4),
    ], axis=1)
    ids = tgt_ids.reshape(bs, 1, t).astype(jnp.int32)

    ct, ioat = pl.pallas_call(
        _matcher_kernel,
        grid=(bs,),
        in_specs=[
            pl.BlockSpec((1, nc, q), lambda b: (b, 0, 0)),
            pl.BlockSpec((1, 4, q), lambda b: (b, 0, 0)),
            pl.BlockSpec((1, npack, 4), lambda b: (b, 0, 0)),
            pl.BlockSpec((1, 1, t), lambda b: (b, 0, 0)),
        ],
        out_specs=[
            pl.BlockSpec((1, t, q), lambda b: (b, 0, 0)),
            pl.BlockSpec((1, ni, q), lambda b: (b, 0, 0)),
        ],
        out_shape=[
            jax.ShapeDtypeStruct((bs, t, q), jnp.float32),
            jax.ShapeDtypeStruct((bs, ni, q), jnp.float32),
        ],
        compiler_params=pltpu.CompilerParams(
            dimension_semantics=("parallel",),
        ),
        name="hungarian_matcher_cost",
        interpret=interpret,
    )(logitsT, boxesT, pack, ids)
    return ct.transpose(0, 2, 1), ioat.transpose(0, 2, 1)


# speedup vs baseline: 1.5686x; 1.0847x over previous
import jax
import jax.numpy as jnp
from jax import lax
from jax.experimental import pallas as pl
from jax.experimental.pallas import tpu as pltpu

_ALPHA, _GAMMA = 0.25, 2.0
_W_CLASS, _W_BBOX, _W_GIOU = 2.0, 5.0, 2.0


def _matcher_kernel(logitsT_ref, boxesT_ref, tgtT_ref, ignT_ref, imgszT_ref,
                    ids_ref, ct_ref, ioat_ref, tgt_scr, ign_scr, ids_scr):
    b = pl.program_id(0)
    nb = pl.num_programs(0)
    logitsT = logitsT_ref[0]
    boxesT = boxesT_ref[0]

    nc, qn = logitsT.shape
    t = ct_ref.shape[1]
    ni = ioat_ref.shape[1]
    tpad = tgt_scr.shape[0] // nb

    eye4 = jnp.where(
        lax.broadcasted_iota(jnp.int32, (4, 4), 0)
        == lax.broadcasted_iota(jnp.int32, (4, 4), 1), 1.0, 0.0)

    @pl.when(b == 0)
    def _():
        dn = (((0,), (0,)), ((), ()))
        tgt_rows = lax.dot_general(
            tgtT_ref[0], eye4, dn, precision=lax.Precision.HIGHEST,
            preferred_element_type=jnp.float32)
        for b2 in range(nb):
            tgt_scr[b2 * tpad:b2 * tpad + t, :] = (
                tgt_rows[b2 * t:(b2 + 1) * t, :])
        ign_scr[...] = lax.dot_general(
            ignT_ref[0], eye4, dn, precision=lax.Precision.HIGHEST,
            preferred_element_type=jnp.float32)
        ids_all = ids_ref[0]
        for b2 in range(nb):
            ids_scr[b2:b2 + 1, 0:t] = ids_all[:, b2 * t:(b2 + 1) * t]

    tgt = tgt_scr[pl.ds(tpad * b, t), :]
    ign = ign_scr[pl.ds(ni * b, ni), :]
    ids = ids_scr[pl.ds(b, 1), 0:t]

    p = jax.nn.sigmoid(logitsT)
    one_m_p = 1.0 - p
    pos = (-_ALPHA) * jnp.log(p + 1e-8) * (one_m_p * one_m_p)
    neg = (-(1.0 - _ALPHA)) * jnp.log(one_m_p + 1e-8) * (p * p)
    cls_cost = pos - neg
    iota_c = lax.broadcasted_iota(jnp.int32, (nc, t), 0)
    onehot = jnp.where(iota_c == ids, _W_CLASS, 0.0)
    cost_class = lax.dot_general(onehot, cls_cost,
                                 (((0,), (0,)), ((), ())),
                                 preferred_element_type=jnp.float32)

    x1 = boxesT[0:1, :]
    y1 = boxesT[1:2, :]
    x2 = boxesT[2:3, :]
    y2 = boxesT[3:4, :]
    tx1 = jnp.broadcast_to(tgt[:, 0:1], (t, qn))
    ty1 = jnp.broadcast_to(tgt[:, 1:2], (t, qn))
    tx2 = jnp.broadcast_to(tgt[:, 2:3], (t, qn))
    ty2 = jnp.broadcast_to(tgt[:, 3:4], (t, qn))

    mxx1 = jnp.maximum(tx1, x1)
    mnx1 = jnp.minimum(tx1, x1)
    mxx2 = jnp.maximum(tx2, x2)
    mnx2 = jnp.minimum(tx2, x2)
    mxy1 = jnp.maximum(ty1, y1)
    mny1 = jnp.minimum(ty1, y1)
    mxy2 = jnp.maximum(ty2, y2)
    mny2 = jnp.minimum(ty2, y2)

    ew = mxx2 - mnx1
    iw_raw = mnx2 - mxx1
    eh = mxy2 - mny1
    ih_raw = mny2 - mxy1
    cost_bbox_w = ew - iw_raw
    cost_bbox_h = eh - ih_raw

    w = imgszT_ref[0, 0, 0]
    h = imgszT_ref[0, 1, 0]
    sw = _W_BBOX / w
    sh = _W_BBOX / h

    inter = jnp.maximum(iw_raw, 0.0) * jnp.maximum(ih_raw, 0.0)
    area_p = (x2 - x1) * (y2 - y1)
    tw = tgt[:, 2:3] - tgt[:, 0:1]
    th = tgt[:, 3:4] - tgt[:, 1:2]
    area_t = tw * th
    union = (area_t + area_p) - inter
    enc = ew * eh
    iou = inter / union
    rest = (enc - union) / enc
    ct_ref[0] = (cost_class + sw * cost_bbox_w + sh * cost_bbox_h
                 + _W_GIOU * (rest - iou))

    ix1 = ign[:, 0:1]
    iy1 = ign[:, 1:2]
    ix2 = ign[:, 2:3]
    iy2 = ign[:, 3:4]
    iiw = jnp.maximum(jnp.minimum(ix2, x2) - jnp.maximum(ix1, x1), 0.0)
    iih = jnp.maximum(jnp.minimum(iy2, y2) - jnp.maximum(iy1, y1), 0.0)
    inv_area = 1.0 / area_p
    ioat_ref[0] = (iiw * iih) * inv_area


def kernel(pred_logits, pred_boxes, tgt_bbox, ign_bbox, image_size_xyxy,
           image_size_xyxy_tgt, tgt_ids, *, interpret=False):
    del image_size_xyxy_tgt
    bs, q, nc = pred_logits.shape
    t = tgt_bbox.shape[0] // bs
    ni = ign_bbox.shape[0] // bs

    logitsT = pred_logits.transpose(0, 2, 1)
    boxesT = pred_boxes.transpose(0, 2, 1)
    tgtT = tgt_bbox.T.reshape(1, 4, bs * t)
    ignT = ign_bbox.T.reshape(1, 4, bs * ni)
    imgszT = image_size_xyxy.T.reshape(1, 4, bs)
    ids = tgt_ids.reshape(1, 1, bs * t).astype(jnp.int32)

    ct, ioat = pl.pallas_call(
        _matcher_kernel,
        grid=(bs,),
        in_specs=[
            pl.BlockSpec((1, nc, q), lambda b: (b, 0, 0)),
            pl.BlockSpec((1, 4, q), lambda b: (b, 0, 0)),
            pl.BlockSpec((1, 4, bs * t), lambda b: (0, 0, 0)),
            pl.BlockSpec((1, 4, bs * ni), lambda b: (0, 0, 0)),
            pl.BlockSpec((1, 4, bs), lambda b: (0, 0, 0)),
            pl.BlockSpec((1, 1, bs * t), lambda b: (0, 0, 0)),
        ],
        out_specs=[
            pl.BlockSpec((1, t, q), lambda b: (b, 0, 0)),
            pl.BlockSpec((1, ni, q), lambda b: (b, 0, 0)),
        ],
        out_shape=[
            jax.ShapeDtypeStruct((bs, t, q), jnp.float32),
            jax.ShapeDtypeStruct((bs, ni, q), jnp.float32),
        ],
        scratch_shapes=[
            pltpu.VMEM((bs * ((t + 7) // 8) * 8, 4), jnp.float32),
            pltpu.VMEM((bs * ni, 4), jnp.float32),
            pltpu.VMEM((bs, 128), jnp.int32),
        ],
        compiler_params=pltpu.CompilerParams(
            dimension_semantics=("arbitrary",),
        ),
        name="hungarian_matcher_cost",
        interpret=interpret,
    )(logitsT, boxesT, tgtT, ignT, imgszT, ids)
    return ct.transpose(0, 2, 1), ioat.transpose(0, 2, 1)
